# baseline (speedup 1.0000x reference)
import functools

import jax
import jax.numpy as jnp
from jax.experimental import pallas as pl
from jax.experimental.pallas import tpu as pltpu


def _moe_kernel(x_ref, v_ref, g_ref, w0_ref, b0_ref, wd_ref, bd_ref,
                out_ref, any_ref, *, n_exp, gate_k, top_k, assume_any):
    x = x_ref[...]
    x16 = x.astype(jnp.bfloat16)
    coef = jnp.dot(x16, v_ref[...], preferred_element_type=jnp.float32)
    cf = coef.astype(jnp.bfloat16).astype(jnp.float32)
    vf = v_ref[...].astype(jnp.float32)
    g = jax.lax.dot_general(x, vf, (((1,), (0,)), ((), ())),
                            precision=jax.lax.Precision.HIGHEST,
                            preferred_element_type=jnp.float32)
    rows = jax.lax.broadcasted_iota(jnp.int32, (n_exp * gate_k, n_exp), 0)
    cols = jax.lax.broadcasted_iota(jnp.int32, (n_exp * gate_k, n_exp), 1)
    seg = (rows // gate_k == cols).astype(jnp.float32)
    xdotpe = jax.lax.dot_general(cf * g, seg, (((1,), (0,)), ((), ())),
                                 precision=jax.lax.Precision.HIGHEST,
                                 preferred_element_type=jnp.float32)
    q = jax.lax.dot_general(cf, g_ref[...], (((1,), (0,)), ((), ())),
                            precision=jax.lax.Precision.HIGHEST,
                            preferred_element_type=jnp.float32)
    pe2 = jax.lax.dot_general(q * cf, seg, (((1,), (0,)), ((), ())),
                              precision=jax.lax.Precision.HIGHEST,
                              preferred_element_type=jnp.float32)
    xn2 = jnp.sum(x * x, axis=1, keepdims=True)
    res = jnp.sqrt(jnp.maximum(xn2 - 2.0 * xdotpe + pe2, 0.0))
    m = jnp.max(-res, axis=1, keepdims=True)
    ex = jnp.exp(-res - m)
    rw = ex / jnp.sum(ex, axis=1, keepdims=True)

    ids = jax.lax.broadcasted_iota(jnp.int32, rw.shape, 1)
    thresh_f = (rw > (1.0 / n_exp)).astype(rw.dtype)
    any_ref[...] = jnp.broadcast_to(jnp.max(thresh_f), any_ref.shape)
    mx1 = jnp.max(rw, axis=1, keepdims=True)
    i1 = jnp.min(jnp.where(rw == mx1, ids, n_exp), axis=1, keepdims=True)
    fb_f = (ids == i1).astype(rw.dtype)
    base_f = thresh_f if assume_any else fb_f
    tk_f = fb_f
    cur = jnp.where(ids == i1, -jnp.inf, rw)
    for _ in range(top_k - 1):
        mxk = jnp.max(cur, axis=1, keepdims=True)
        ik = jnp.min(jnp.where(cur == mxk, ids, n_exp), axis=1, keepdims=True)
        tk_f = tk_f + (ids == ik).astype(rw.dtype)
        cur = jnp.where(ids == ik, -jnp.inf, cur)
    filt = rw * base_f * tk_f
    sw = jnp.sum(filt, axis=1, keepdims=True)
    sw = jnp.where(sw == 0.0, 1.0, sw)
    nw = filt / sw

    dn = (((1,), (1,)), ((), ()))
    acc = jax.lax.dot_general(x16, w0_ref[...], dn,
                              preferred_element_type=jnp.float32)
    acc = acc + b0_ref[...]
    acc = acc + jnp.dot(nw, bd_ref[...], preferred_element_type=jnp.float32)
    for e in range(n_exp):
        pe = jax.lax.dot_general(x16, wd_ref[e], dn,
                                 preferred_element_type=jnp.float32)
        acc = acc + nw[:, e:e + 1] * pe
    out_ref[...] = acc


def kernel(hidden_states, W0, b0, Wdiff, bdiff, orig_v):
    B, S, D_IN = hidden_states.shape
    E, D_OUT, _ = Wdiff.shape
    GK = orig_v.shape[2]
    TOP_K = 2
    T = B * S
    TB = 1024
    NB = T // TB

    x = hidden_states.reshape(T, D_IN)
    v_flat = jnp.transpose(orig_v, (1, 0, 2)).reshape(D_IN, E * GK)
    v16 = v_flat.astype(jnp.bfloat16)
    vgf = v16.astype(jnp.float32)
    gram = jnp.einsum('dr,ds->rs', vgf, vgf,
                      precision=jax.lax.Precision.HIGHEST)
    rblk = jnp.arange(E * GK)[:, None] // GK
    gram = gram * (rblk == rblk.T).astype(jnp.float32)
    w016 = W0.astype(jnp.bfloat16)
    wd16 = Wdiff.astype(jnp.bfloat16)
    b0r = b0.reshape(1, D_OUT)

    def run(assume_any):
        return pl.pallas_call(
            functools.partial(_moe_kernel, n_exp=E, gate_k=GK, top_k=TOP_K,
                              assume_any=assume_any),
            grid=(NB,),
            in_specs=[
                pl.BlockSpec((TB, D_IN), lambda i: (i, 0)),
                pl.BlockSpec((D_IN, E * GK), lambda i: (0, 0)),
                pl.BlockSpec((E * GK, E * GK), lambda i: (0, 0)),
                pl.BlockSpec((D_OUT, D_IN), lambda i: (0, 0)),
                pl.BlockSpec((1, D_OUT), lambda i: (0, 0)),
                pl.BlockSpec((E, D_OUT, D_IN), lambda i: (0, 0, 0)),
                pl.BlockSpec((E, D_OUT), lambda i: (0, 0)),
            ],
            out_specs=[
                pl.BlockSpec((TB, D_OUT), lambda i: (i, 0)),
                pl.BlockSpec((1, 1, 128), lambda i: (i, 0, 0)),
            ],
            out_shape=[
                jax.ShapeDtypeStruct((T, D_OUT), jnp.float32),
                jax.ShapeDtypeStruct((NB, 1, 128), jnp.float32),
            ],
            compiler_params=pltpu.CompilerParams(
                dimension_semantics=("parallel",)),
        )(x, v16, gram, w016, b0r, wd16, bdiff)

    out_main, any_arr = run(True)
    any_flag = jnp.max(any_arr) > 0.0
    out = jax.lax.cond(any_flag,
                       lambda: out_main,
                       lambda: run(False)[0])
    return out.reshape(B, S, D_OUT)


# speedup vs baseline: 1.3605x; 1.3605x over previous
import functools

import jax
import jax.numpy as jnp
from jax.experimental import pallas as pl
from jax.experimental.pallas import tpu as pltpu


def _moe_kernel(x_ref, v_ref, g_ref, w0_ref, b0_ref, wd_ref, bd_ref,
                out_ref, any_ref, *, n_exp, gate_k, top_k, assume_any):
    dnc = (((1,), (0,)), ((), ()))

    def _dot16(a16, b16):
        return jax.lax.dot_general(a16, b16, dnc,
                                   preferred_element_type=jnp.float32)

    def _split_dot(z, b16):
        zh = z.astype(jnp.bfloat16)
        zl = (z - zh.astype(jnp.float32)).astype(jnp.bfloat16)
        return _dot16(zh, b16) + _dot16(zl, b16)

    x = x_ref[...]
    x16 = x.astype(jnp.bfloat16)
    v16 = v_ref[...]
    coef = _dot16(x16, v16)
    c16 = coef.astype(jnp.bfloat16)
    cf = c16.astype(jnp.float32)
    dx16 = (x - x16.astype(jnp.float32)).astype(jnp.bfloat16)
    g = coef + _dot16(dx16, v16)
    rows = jax.lax.broadcasted_iota(jnp.int32, (n_exp * gate_k, n_exp), 0)
    cols = jax.lax.broadcasted_iota(jnp.int32, (n_exp * gate_k, n_exp), 1)
    seg16 = (rows // gate_k == cols).astype(jnp.bfloat16)
    xdotpe = _split_dot(cf * g, seg16)
    q = _dot16(c16, g_ref[0]) + _dot16(c16, g_ref[1])
    pe2 = _split_dot(q * cf, seg16)
    xn2 = jnp.sum(x * x, axis=1, keepdims=True)
    res = jnp.sqrt(jnp.maximum(xn2 - 2.0 * xdotpe + pe2, 0.0))
    m = jnp.max(-res, axis=1, keepdims=True)
    ex = jnp.exp(-res - m)
    rw = ex / jnp.sum(ex, axis=1, keepdims=True)

    ids = jax.lax.broadcasted_iota(jnp.int32, rw.shape, 1)
    thresh_f = (rw > (1.0 / n_exp)).astype(rw.dtype)
    any_ref[...] = jnp.broadcast_to(jnp.max(thresh_f), any_ref.shape)
    mx1 = jnp.max(rw, axis=1, keepdims=True)
    i1 = jnp.min(jnp.where(rw == mx1, ids, n_exp), axis=1, keepdims=True)
    fb_f = (ids == i1).astype(rw.dtype)
    base_f = thresh_f if assume_any else fb_f
    tk_f = fb_f
    cur = jnp.where(ids == i1, -jnp.inf, rw)
    for _ in range(top_k - 1):
        mxk = jnp.max(cur, axis=1, keepdims=True)
        ik = jnp.min(jnp.where(cur == mxk, ids, n_exp), axis=1, keepdims=True)
        tk_f = tk_f + (ids == ik).astype(rw.dtype)
        cur = jnp.where(ids == ik, -jnp.inf, cur)
    filt = rw * base_f * tk_f
    sw = jnp.sum(filt, axis=1, keepdims=True)
    sw = jnp.where(sw == 0.0, 1.0, sw)
    nw = filt / sw

    dn = (((1,), (1,)), ((), ()))
    acc = jax.lax.dot_general(x16, w0_ref[...], dn,
                              preferred_element_type=jnp.float32)
    acc = acc + b0_ref[...]
    acc = acc + jnp.dot(nw, bd_ref[...], preferred_element_type=jnp.float32)
    for e in range(n_exp):
        pe = jax.lax.dot_general(x16, wd_ref[e], dn,
                                 preferred_element_type=jnp.float32)
        acc = acc + nw[:, e:e + 1] * pe
    out_ref[...] = acc


def kernel(hidden_states, W0, b0, Wdiff, bdiff, orig_v):
    B, S, D_IN = hidden_states.shape
    E, D_OUT, _ = Wdiff.shape
    GK = orig_v.shape[2]
    TOP_K = 2
    T = B * S
    TB = 1024
    NB = T // TB

    x = hidden_states.reshape(T, D_IN)
    v_flat = jnp.transpose(orig_v, (1, 0, 2)).reshape(D_IN, E * GK)
    v16 = v_flat.astype(jnp.bfloat16)
    vgf = v16.astype(jnp.float32)
    gram = jnp.einsum('dr,ds->rs', vgf, vgf,
                      precision=jax.lax.Precision.HIGHEST)
    rblk = jnp.arange(E * GK)[:, None] // GK
    gram = gram * (rblk == rblk.T).astype(jnp.float32)
    gram_hi = gram.astype(jnp.bfloat16)
    gram_lo = (gram - gram_hi.astype(jnp.float32)).astype(jnp.bfloat16)
    gram2 = jnp.stack([gram_hi, gram_lo])
    w016 = W0.astype(jnp.bfloat16)
    wd16 = Wdiff.astype(jnp.bfloat16)
    b0r = b0.reshape(1, D_OUT)

    def run(assume_any):
        return pl.pallas_call(
            functools.partial(_moe_kernel, n_exp=E, gate_k=GK, top_k=TOP_K,
                              assume_any=assume_any),
            grid=(NB,),
            in_specs=[
                pl.BlockSpec((TB, D_IN), lambda i: (i, 0)),
                pl.BlockSpec((D_IN, E * GK), lambda i: (0, 0)),
                pl.BlockSpec((2, E * GK, E * GK), lambda i: (0, 0, 0)),
                pl.BlockSpec((D_OUT, D_IN), lambda i: (0, 0)),
                pl.BlockSpec((1, D_OUT), lambda i: (0, 0)),
                pl.BlockSpec((E, D_OUT, D_IN), lambda i: (0, 0, 0)),
                pl.BlockSpec((E, D_OUT), lambda i: (0, 0)),
            ],
            out_specs=[
                pl.BlockSpec((TB, D_OUT), lambda i: (i, 0)),
                pl.BlockSpec((1, 1, 128), lambda i: (i, 0, 0)),
            ],
            out_shape=[
                jax.ShapeDtypeStruct((T, D_OUT), jnp.float32),
                jax.ShapeDtypeStruct((NB, 1, 128), jnp.float32),
            ],
            compiler_params=pltpu.CompilerParams(
                dimension_semantics=("parallel",)),
        )(x, v16, gram2, w016, b0r, wd16, bdiff)

    out_main, any_arr = run(True)
    any_flag = jnp.max(any_arr) > 0.0
    out = jax.lax.cond(any_flag,
                       lambda: out_main,
                       lambda: run(False)[0])
    return out.reshape(B, S, D_OUT)
